# baseline (speedup 1.0000x reference)
import jax
import jax.numpy as jnp
from jax.experimental import pallas as pl
from jax.experimental.pallas import tpu as pltpu

BATCH = 4
SEQ = 8192
D = 1024

ROWS = 512
SCHUNKS = SEQ // ROWS
STEPS = SCHUNKS * BATCH


def _body(x_hbm, pos_hbm, o_hbm, xb, pb, in_sems, out_sems, pos_sems):
    def start_x(step):
        c, b = step // BATCH, step % BATCH
        p = step % 4
        return pltpu.make_async_copy(
            x_hbm.at[b, pl.ds(c * ROWS, ROWS), :], xb.at[p],
            in_sems.at[p])

    def start_pos(c):
        p = c % 2
        return pltpu.make_async_copy(
            pos_hbm.at[pl.ds(c * ROWS, ROWS), :], pb.at[p], pos_sems.at[p])

    def start_out(step):
        c, b = step // BATCH, step % BATCH
        p = step % 4
        return pltpu.make_async_copy(
            xb.at[p], o_hbm.at[b, pl.ds(c * ROWS, ROWS), :], out_sems.at[p])

    in_flight = {}
    pos_flight = {}
    out_flight = {}

    pos_flight[0] = start_pos(0)
    pos_flight[0].start()
    for s0 in (0, 1):
        in_flight[s0] = start_x(s0)
        in_flight[s0].start()

    for s in range(STEPS):
        p = s % 4
        c = s // BATCH
        nxt = s + 2
        if nxt < STEPS:
            if nxt - 4 >= 0:
                out_flight[nxt - 4].wait()
            in_flight[nxt] = start_x(nxt)
            in_flight[nxt].start()
        if s % BATCH == 3 and c + 1 < SCHUNKS:
            pos_flight[c + 1] = start_pos(c + 1)
            pos_flight[c + 1].start()

        in_flight[s].wait()
        if s % BATCH == 0:
            pos_flight[c].wait()

        xb[p] = xb[p] + pb[c % 2]

        out_flight[s] = start_out(s)
        out_flight[s].start()

    for s in range(STEPS - 4, STEPS):
        out_flight[s].wait()


def kernel(x, pos_table):
    return pl.pallas_call(
        _body,
        in_specs=[
            pl.BlockSpec(memory_space=pl.ANY),
            pl.BlockSpec(memory_space=pl.ANY),
        ],
        out_specs=pl.BlockSpec(memory_space=pl.ANY),
        out_shape=jax.ShapeDtypeStruct(x.shape, x.dtype),
        scratch_shapes=[
            pltpu.VMEM((4, ROWS, D), jnp.float32),
            pltpu.VMEM((2, ROWS, D), jnp.float32),
            pltpu.SemaphoreType.DMA((4,)),
            pltpu.SemaphoreType.DMA((4,)),
            pltpu.SemaphoreType.DMA((2,)),
        ],
    )(x, pos_table)


# speedup vs baseline: 1.1289x; 1.1289x over previous
import jax
import jax.numpy as jnp
from jax.experimental import pallas as pl
from jax.experimental.pallas import tpu as pltpu

BATCH = 4
SEQ = 8192
D = 1024

ROWS = 2048
SCHUNKS = SEQ // ROWS
STEPS = SCHUNKS * BATCH


def _body(x_hbm, pos_hbm, o_hbm, xb, pb, in_sems, out_sems, pos_sems):
    def start_x(step):
        c, b = step // BATCH, step % BATCH
        p = step % 4
        return pltpu.make_async_copy(
            x_hbm.at[b, pl.ds(c * ROWS, ROWS), :], xb.at[p],
            in_sems.at[p])

    def start_pos(c):
        p = c % 2
        return pltpu.make_async_copy(
            pos_hbm.at[pl.ds(c * ROWS, ROWS), :], pb.at[p], pos_sems.at[p])

    def start_out(step):
        c, b = step // BATCH, step % BATCH
        p = step % 4
        return pltpu.make_async_copy(
            xb.at[p], o_hbm.at[b, pl.ds(c * ROWS, ROWS), :], out_sems.at[p])

    in_flight = {}
    pos_flight = {}
    out_flight = {}

    pos_flight[0] = start_pos(0)
    pos_flight[0].start()
    for s0 in (0, 1):
        in_flight[s0] = start_x(s0)
        in_flight[s0].start()

    for s in range(STEPS):
        p = s % 4
        c = s // BATCH
        nxt = s + 2
        if nxt < STEPS:
            if nxt - 4 >= 0:
                out_flight[nxt - 4].wait()
            in_flight[nxt] = start_x(nxt)
            in_flight[nxt].start()
        if s % BATCH == 3 and c + 1 < SCHUNKS:
            pos_flight[c + 1] = start_pos(c + 1)
            pos_flight[c + 1].start()

        in_flight[s].wait()
        if s % BATCH == 0:
            pos_flight[c].wait()

        xb[p] = xb[p] + pb[c % 2]

        out_flight[s] = start_out(s)
        out_flight[s].start()

    for s in range(STEPS - 4, STEPS):
        out_flight[s].wait()


def kernel(x, pos_table):
    return pl.pallas_call(
        _body,
        in_specs=[
            pl.BlockSpec(memory_space=pl.ANY),
            pl.BlockSpec(memory_space=pl.ANY),
        ],
        out_specs=pl.BlockSpec(memory_space=pl.ANY),
        out_shape=jax.ShapeDtypeStruct(x.shape, x.dtype),
        scratch_shapes=[
            pltpu.VMEM((4, ROWS, D), jnp.float32),
            pltpu.VMEM((2, ROWS, D), jnp.float32),
            pltpu.SemaphoreType.DMA((4,)),
            pltpu.SemaphoreType.DMA((4,)),
            pltpu.SemaphoreType.DMA((2,)),
        ],
    )(x, pos_table)
